# baseline (speedup 1.0000x reference)
import functools

import jax
import jax.numpy as jnp
from jax import lax
from jax.experimental import pallas as pl
from jax.experimental.pallas import tpu as pltpu
from jax.experimental.pallas import tpu_sc as plsc

L = 16
SPLIT = 2


def kernel(h, indicies, emb_weight):
    B, _, D = h.shape
    S = indicies.shape[1]
    NW = 32
    RPW = B // NW
    SS = S // SPLIT

    idx = indicies.astype(jnp.int32).reshape(NW, RPW, SPLIT, SS)
    hw = h.reshape(NW, RPW, D)

    mesh = plsc.VectorSubcoreMesh(core_axis_name="c", subcore_axis_name="s")

    @functools.partial(
        pl.kernel,
        out_type=jax.ShapeDtypeStruct((NW, RPW, S), jnp.float32),
        mesh=mesh,
        compiler_params=pltpu.CompilerParams(
            needs_layout_passes=False, use_tc_tiling_on_sc=False
        ),
        scratch_types=[
            pltpu.VMEM((RPW, SPLIT, SS), jnp.int32),
            pltpu.VMEM((RPW, D), jnp.float32),
            pltpu.VMEM((RPW, S), jnp.float32),
            pltpu.VMEM((S, D), jnp.float32),
            pltpu.SemaphoreType.DMA,
        ],
    )
    def run(idx_hbm, h_hbm, tbl_hbm, out_hbm, idx_v, h_v, out_v, rows_v, sem):
        wid = lax.axis_index("s") * 2 + lax.axis_index("c")
        pltpu.sync_copy(idx_hbm.at[wid], idx_v)
        pltpu.sync_copy(h_hbm.at[wid], h_v)

        lane = lax.iota(jnp.int32, L)
        ngroups = (S + L - 1) // L

        @pl.loop(0, RPW)
        def row_loop(r):
            cps = [
                pltpu.async_copy(
                    tbl_hbm.at[idx_v.at[r, j]],
                    rows_v.at[pl.ds(j * SS, SS)],
                    sem,
                )
                for j in range(SPLIT)
            ]
            for c in cps:
                c.wait()

            hvecs = [h_v[r, pl.ds(q * L, L)] for q in range(D // L)]

            @pl.loop(0, ngroups)
            def s_loop(g):
                s_base = jnp.minimum(g * L, S - L)
                sidx = lane + s_base
                accs = [None] * 4
                for d in range(D):
                    dv = jnp.full((L,), d, jnp.int32)
                    wv = plsc.load_gather(rows_v, [sidx, dv])
                    term = wv * hvecs[d // L][d % L]
                    q = d % 4
                    accs[q] = term if accs[q] is None else accs[q] + term
                out_v[r, pl.ds(s_base, L)] = (accs[0] + accs[1]) + (
                    accs[2] + accs[3]
                )

        pltpu.sync_copy(out_v, out_hbm.at[wid])

    out = run(idx, hw, emb_weight)
    return out.reshape(B, 1, S)


# speedup vs baseline: 1.0686x; 1.0686x over previous
import functools

import jax
import jax.numpy as jnp
from jax import lax
from jax.experimental import pallas as pl
from jax.experimental.pallas import tpu as pltpu
from jax.experimental.pallas import tpu_sc as plsc

L = 16
SPLIT = 2
NBUF = 4


def kernel(h, indicies, emb_weight):
    B, _, D = h.shape
    S = indicies.shape[1]
    NW = 32
    RPW = B // NW
    SS = S // SPLIT

    idx = indicies.astype(jnp.int32).reshape(NW, RPW, SPLIT, SS)
    hw = h.reshape(NW, RPW, D)

    mesh = plsc.VectorSubcoreMesh(core_axis_name="c", subcore_axis_name="s")

    @functools.partial(
        pl.kernel,
        out_type=jax.ShapeDtypeStruct((NW, RPW, S), jnp.float32),
        mesh=mesh,
        compiler_params=pltpu.CompilerParams(
            needs_layout_passes=False, use_tc_tiling_on_sc=False
        ),
        scratch_types=[
            pltpu.VMEM((RPW, SPLIT, SS), jnp.int32),
            pltpu.VMEM((RPW, D), jnp.float32),
            pltpu.VMEM((RPW, S), jnp.float32),
            pltpu.VMEM((NBUF, S, D), jnp.float32),
            [pltpu.SemaphoreType.DMA] * NBUF,
        ],
    )
    def run(idx_hbm, h_hbm, tbl_hbm, out_hbm, idx_v, h_v, out_v, rows_v, sems):
        wid = lax.axis_index("s") * 2 + lax.axis_index("c")
        pltpu.sync_copy(idx_hbm.at[wid], idx_v)
        pltpu.sync_copy(h_hbm.at[wid], h_v)

        lane = lax.iota(jnp.int32, L)
        ngroups = (S + L - 1) // L

        def fire(r, b):
            for j in range(SPLIT):
                pltpu.async_copy(
                    tbl_hbm.at[idx_v.at[r, j]],
                    rows_v.at[b, pl.ds(j * SS, SS)],
                    sems[b],
                )

        def drain(b):
            pltpu.make_async_copy(
                tbl_hbm.at[pl.ds(0, S)], rows_v.at[b], sems[b]
            ).wait()

        for b in range(NBUF - 1):
            fire(b, b)

        @pl.loop(0, RPW // NBUF)
        def row_loop(rp):
            for k in range(NBUF):
                r = rp * NBUF + k
                drain(k)

                @pl.when(r + NBUF - 1 < RPW)
                def _():
                    fire(r + NBUF - 1, (k + NBUF - 1) % NBUF)

                hvecs = [h_v[r, pl.ds(q * L, L)] for q in range(D // L)]

                @pl.loop(0, ngroups)
                def s_loop(g):
                    s_base = jnp.minimum(g * L, S - L)
                    sidx = lane + s_base
                    accs = [None] * 4
                    for d in range(D):
                        dv = jnp.full((L,), d, jnp.int32)
                        wv = plsc.load_gather(rows_v.at[k], [sidx, dv])
                        term = wv * hvecs[d // L][d % L]
                        q = d % 4
                        accs[q] = term if accs[q] is None else accs[q] + term
                    out_v[r, pl.ds(s_base, L)] = (accs[0] + accs[1]) + (
                        accs[2] + accs[3]
                    )

        pltpu.sync_copy(out_v, out_hbm.at[wid])

    out = run(idx, hw, emb_weight)
    return out.reshape(B, 1, S)


# speedup vs baseline: 1.5187x; 1.4213x over previous
import functools

import jax
import jax.numpy as jnp
from jax import lax
from jax.experimental import pallas as pl
from jax.experimental.pallas import tpu as pltpu
from jax.experimental.pallas import tpu_sc as plsc

L = 16
SPLIT = 2
NBUF = 4


def kernel(h, indicies, emb_weight):
    B, _, D = h.shape
    S = indicies.shape[1]
    NW = 32
    RPW = B // NW
    SS = S // SPLIT

    idx = indicies.astype(jnp.int32).reshape(NW, RPW, SPLIT, SS)
    hw = h.reshape(NW, RPW, D)

    mesh = plsc.VectorSubcoreMesh(core_axis_name="c", subcore_axis_name="s")

    @functools.partial(
        pl.kernel,
        out_type=jax.ShapeDtypeStruct((NW, RPW * S), jnp.float32),
        mesh=mesh,
        compiler_params=pltpu.CompilerParams(
            needs_layout_passes=False, use_tc_tiling_on_sc=False
        ),
        scratch_types=[
            pltpu.VMEM((RPW, SPLIT, SS), jnp.int32),
            pltpu.VMEM((RPW, D), jnp.float32),
            pltpu.VMEM((RPW * S,), jnp.float32),
            pltpu.VMEM((NBUF, S, D), jnp.float32),
            [pltpu.SemaphoreType.DMA] * NBUF,
        ],
    )
    def run(idx_hbm, h_hbm, tbl_hbm, out_hbm, idx_v, h_v, out_v, rows_v, sems):
        wid = lax.axis_index("s") * 2 + lax.axis_index("c")
        pltpu.sync_copy(idx_hbm.at[wid], idx_v)
        pltpu.sync_copy(h_hbm.at[wid], h_v)

        lane = lax.iota(jnp.int32, L)
        last_mask = lane == (L - 1)
        ngroups = (S + L - 1) // L

        def fire(r, b):
            for j in range(SPLIT):
                pltpu.async_copy(
                    tbl_hbm.at[idx_v.at[r, j]],
                    rows_v.at[b, pl.ds(j * SS, SS)],
                    sems[b],
                )

        def drain(b):
            pltpu.make_async_copy(
                tbl_hbm.at[pl.ds(0, S)], rows_v.at[b], sems[b]
            ).wait()

        for b in range(NBUF - 1):
            fire(b, b)

        @pl.loop(0, RPW // NBUF)
        def row_loop(rp):
            for k in range(NBUF):
                r = rp * NBUF + k
                drain(k)

                @pl.when(r + NBUF - 1 < RPW)
                def _():
                    fire(r + NBUF - 1, (k + NBUF - 1) % NBUF)

                hv = [h_v[r, pl.ds(q * L, L)] for q in range(D // L)]

                @pl.loop(0, ngroups)
                def s_loop(g):
                    s_base = jnp.minimum(g * L, S - L)
                    fvec = r * S + s_base - (L - 1) + lane

                    for k2 in range(L):
                        row = rows_v.at[k]
                        s = s_base + k2
                        acc = row[s, pl.ds(0, L)] * hv[0]
                        acc = acc + row[s, pl.ds(L, L)] * hv[1]
                        acc = acc + row[s, pl.ds(2 * L, L)] * hv[2]
                        acc = acc + row[s, pl.ds(3 * L, L)] * hv[3]
                        plsc.store_scatter(
                            out_v,
                            [fvec + k2],
                            plsc.cumsum(acc),
                            mask=last_mask,
                        )

        pltpu.sync_copy(out_v, out_hbm.at[wid])

    out = run(idx, hw, emb_weight)
    return out.reshape(B, 1, S)


# speedup vs baseline: 2.2113x; 1.4560x over previous
import functools

import jax
import jax.numpy as jnp
from jax import lax
from jax.experimental import pallas as pl
from jax.experimental.pallas import tpu as pltpu
from jax.experimental.pallas import tpu_sc as plsc

L = 16
SPLIT = 2
NBUF = 4


def kernel(h, indicies, emb_weight):
    B, _, D = h.shape
    S = indicies.shape[1]
    NW = 32
    RPW = B // NW
    SS = S // SPLIT
    NG = (S + L - 1) // L
    s_bases = [min(g * L, S - L) for g in range(NG)]

    idx = indicies.astype(jnp.int32).reshape(NW, RPW, SPLIT, SS)
    h2 = h.reshape(B, D)
    hdup = jnp.concatenate([h2, h2[:, :L]], axis=1).reshape(NW, RPW, D + L)

    mesh = plsc.VectorSubcoreMesh(core_axis_name="c", subcore_axis_name="s")

    @functools.partial(
        pl.kernel,
        out_type=jax.ShapeDtypeStruct((NW, RPW * S), jnp.float32),
        mesh=mesh,
        compiler_params=pltpu.CompilerParams(
            needs_layout_passes=False, use_tc_tiling_on_sc=False
        ),
        scratch_types=[
            pltpu.VMEM((RPW, SPLIT, SS), jnp.int32),
            pltpu.VMEM((RPW, D + L), jnp.float32),
            pltpu.VMEM((RPW * S,), jnp.float32),
            pltpu.VMEM((NBUF, S, D), jnp.float32),
            [pltpu.SemaphoreType.DMA] * NBUF,
        ],
    )
    def run(idx_hbm, h_hbm, tbl_hbm, out_hbm, idx_v, h_v, out_v, rows_v, sems):
        wid = lax.axis_index("s") * 2 + lax.axis_index("c")
        pltpu.sync_copy(idx_hbm.at[wid], idx_v)
        pltpu.sync_copy(h_hbm.at[wid], h_v)

        lane = lax.iota(jnp.int32, L)
        sidx = [sb + lane for sb in s_bases]

        def fire(r, b):
            for j in range(SPLIT):
                pltpu.async_copy(
                    tbl_hbm.at[idx_v.at[r, j]],
                    rows_v.at[b, pl.ds(j * SS, SS)],
                    sems[b],
                )

        def drain(b):
            pltpu.make_async_copy(
                tbl_hbm.at[pl.ds(0, S)], rows_v.at[b], sems[b]
            ).wait()

        for b in range(NBUF - 1):
            fire(b, b)

        @pl.loop(0, RPW // NBUF)
        def row_loop(rp):
            for k in range(NBUF):
                r = rp * NBUF + k
                drain(k)

                @pl.when(r + NBUF - 1 < RPW)
                def _():
                    fire(r + NBUF - 1, (k + NBUF - 1) % NBUF)

                init = (jnp.zeros((L,), jnp.float32),) * NG

                @pl.loop(0, D, init_carry=init)
                def d_loop(d, accs):
                    dvec = lane + d
                    dvec = jnp.where(dvec >= D, dvec - D, dvec)
                    hrot = h_v[r, pl.ds(d, L)]
                    return tuple(
                        accs[g]
                        + plsc.load_gather(rows_v.at[k], [sidx[g], dvec])
                        * hrot
                        for g in range(NG)
                    )

                for g in range(NG):
                    out_v[pl.ds(r * S + s_bases[g], L)] = d_loop[g]

        pltpu.sync_copy(out_v, out_hbm.at[wid])

    out = run(idx, hdup, emb_weight)
    return out.reshape(B, 1, S)
